# baseline (device time: 17450 ns/iter reference)
import jax
import jax.numpy as jnp
from jax import lax
from jax.experimental import pallas as pl
from jax.experimental.pallas import tpu as pltpu

N_DEV = 4
COMM_DTYPE = jnp.bfloat16

RAW_A, RAW_B, DIR_A, DIR_B, MRG_A, MRG_B = range(6)


def kernel(x, w_mat):
    m_full, _ = x.shape
    _, n = w_mat.shape
    m_chunk = m_full // N_DEV
    nh = n // 2

    def body(x_ref, w_ref, out_ref, mown_a, mown_b, pown,
             sbuf, rbuf, ssem, rsem):
        my = lax.axis_index("i")
        left = lax.rem(my + N_DEV - 1, N_DEV)
        right = lax.rem(my + 1, N_DEV)

        def rows(c):
            return pl.ds(lax.rem(c + 2 * N_DEV, N_DEV) * m_chunk, m_chunk)

        def half_dot(c, cols):
            return jnp.dot(
                x_ref[rows(c), :], w_ref[:, cols],
                preferred_element_type=jnp.float32,
            )

        def send(flow, dev):
            d = pltpu.make_async_remote_copy(
                src_ref=sbuf.at[flow], dst_ref=rbuf.at[flow],
                send_sem=ssem.at[flow], recv_sem=rsem.at[flow],
                device_id=(dev,), device_id_type=pl.DeviceIdType.MESH,
            )
            d.start()
            return d

        barrier_sem = pltpu.get_barrier_semaphore()
        for nbr in (left, right):
            pl.semaphore_signal(
                barrier_sem, inc=1,
                device_id=(nbr,), device_id_type=pl.DeviceIdType.MESH,
            )
        pl.semaphore_wait(barrier_sem, 2)

        descs = {}

        sbuf[RAW_A, :, :] = half_dot(my + 2, slice(0, nh)).astype(COMM_DTYPE)
        descs[RAW_A] = send(RAW_A, left)
        sbuf[RAW_B, :, :] = half_dot(my + 2, slice(nh, n)).astype(COMM_DTYPE)
        descs[RAW_B] = send(RAW_B, right)

        sbuf[DIR_A, :, :] = half_dot(my + 1, slice(0, nh)).astype(COMM_DTYPE)
        descs[DIR_A] = send(DIR_A, right)
        sbuf[DIR_B, :, :] = half_dot(my - 1, slice(nh, n)).astype(COMM_DTYPE)
        descs[DIR_B] = send(DIR_B, left)

        mown_a[:, :] = half_dot(my - 1, slice(0, nh))
        mown_b[:, :] = half_dot(my + 1, slice(nh, n))

        descs[RAW_A].wait_recv()
        sbuf[MRG_A, :, :] = (
            mown_a[:, :] + rbuf[RAW_A].astype(jnp.float32)
        ).astype(COMM_DTYPE)
        descs[MRG_A] = send(MRG_A, left)
        descs[RAW_B].wait_recv()
        sbuf[MRG_B, :, :] = (
            mown_b[:, :] + rbuf[RAW_B].astype(jnp.float32)
        ).astype(COMM_DTYPE)
        descs[MRG_B] = send(MRG_B, right)

        pown[:, :] = jnp.dot(
            x_ref[rows(my), :], w_ref[:, :],
            preferred_element_type=jnp.float32,
        )

        descs[DIR_A].wait_recv()
        descs[MRG_A].wait_recv()
        out_ref[:, :nh] = (
            pown[:, :nh]
            + rbuf[DIR_A].astype(jnp.float32)
            + rbuf[MRG_A].astype(jnp.float32)
        )
        descs[DIR_B].wait_recv()
        descs[MRG_B].wait_recv()
        out_ref[:, nh:] = (
            pown[:, nh:]
            + rbuf[DIR_B].astype(jnp.float32)
            + rbuf[MRG_B].astype(jnp.float32)
        )

        for d in descs.values():
            d.wait_send()

    comm_shape = (6, m_chunk, nh)
    return pl.pallas_call(
        body,
        out_shape=jax.ShapeDtypeStruct((m_chunk, n), jnp.float32),
        in_specs=[
            pl.BlockSpec(memory_space=pltpu.VMEM),
            pl.BlockSpec(memory_space=pltpu.VMEM),
        ],
        out_specs=pl.BlockSpec(memory_space=pltpu.VMEM),
        scratch_shapes=[
            pltpu.VMEM((m_chunk, nh), jnp.float32),
            pltpu.VMEM((m_chunk, nh), jnp.float32),
            pltpu.VMEM((m_chunk, n), jnp.float32),
            pltpu.VMEM(comm_shape, COMM_DTYPE),
            pltpu.VMEM(comm_shape, COMM_DTYPE),
            pltpu.SemaphoreType.DMA((6,)),
            pltpu.SemaphoreType.DMA((6,)),
        ],
        compiler_params=pltpu.CompilerParams(collective_id=0),
    )(x, w_mat)
